# baseline (speedup 1.0000x reference)
import functools

import jax
import jax.numpy as jnp
from jax import lax
from jax.experimental import pallas as pl
from jax.experimental.pallas import tpu as pltpu
from jax.experimental.pallas import tpu_sc as plsc

EMB = 64
MAX_LEN = 200
NUM_WORKERS = 32
LANES = 16
CHUNK = 800


def _body(idx_hbm, table_hbm, pos_hbm, out_hbm, idx_v, rows_v, pos_v, sem):
    per_w = idx_hbm.shape[0] // NUM_WORKERS
    nchunk = per_w // CHUNK
    reps = CHUNK // MAX_LEN

    wid = lax.axis_index("s") * 2 + lax.axis_index("c")
    base = wid * per_w

    pltpu.sync_copy(pos_hbm, pos_v)

    def chunk_body(k, _):
        off = base + k * CHUNK
        pltpu.sync_copy(idx_hbm.at[pl.ds(off, CHUNK)], idx_v)
        pltpu.async_copy(table_hbm.at[idx_v], rows_v, sem).wait()

        def add_l(l, carry):
            pvals = [pos_v[l, pl.ds(16 * j, 16)] for j in range(EMB // LANES)]
            for r in range(reps):
                i = r * MAX_LEN + l
                for j in range(EMB // LANES):
                    sl = pl.ds(16 * j, 16)
                    rows_v[i, sl] = rows_v[i, sl] + pvals[j]
            return carry

        lax.fori_loop(0, MAX_LEN, add_l, 0)

        pltpu.sync_copy(rows_v, out_hbm.at[pl.ds(off, CHUNK)])
        return 0

    lax.fori_loop(0, nchunk, chunk_body, 0)


def kernel(to_emb, token_table, pos_table):
    batch, seq_len = to_emb.shape
    flat = batch * seq_len
    idx = to_emb.reshape(flat).astype(jnp.int32)

    mesh = plsc.VectorSubcoreMesh(core_axis_name="c", subcore_axis_name="s")
    k = functools.partial(
        pl.kernel,
        out_type=jax.ShapeDtypeStruct((flat, EMB), jnp.float32),
        mesh=mesh,
        scratch_types=[
            pltpu.VMEM((CHUNK,), jnp.int32),
            pltpu.VMEM((CHUNK, EMB), jnp.float32),
            pltpu.VMEM((MAX_LEN, EMB), jnp.float32),
            pltpu.SemaphoreType.DMA,
        ],
        compiler_params=pltpu.CompilerParams(use_tc_tiling_on_sc=False),
    )(_body)
    out = k(idx, token_table, pos_table)
    return out.reshape(batch, seq_len, EMB)
